# baseline (speedup 1.0000x reference)
import functools

import jax
import jax.numpy as jnp
from jax import lax
from jax.experimental import pallas as pl
from jax.experimental.pallas import tpu as pltpu
from jax.experimental.pallas import tpu_sc as plsc


def _gather_pe(t, pe_weight):
    (n,) = t.shape
    period, c = pe_weight.shape
    info = plsc.get_sparse_core_info()
    nw = info.num_cores * info.num_subcores
    n_per_w = n // nw
    mesh = plsc.VectorSubcoreMesh(core_axis_name="c", subcore_axis_name="s")

    @functools.partial(
        pl.kernel,
        mesh=mesh,
        out_type=jax.ShapeDtypeStruct((n, c), jnp.float32),
        scratch_types=[
            pltpu.VMEM((n_per_w,), jnp.int32),
            pltpu.VMEM((n_per_w, c), jnp.float32),
            pltpu.SemaphoreType.DMA,
        ],
        compiler_params=pltpu.CompilerParams(use_tc_tiling_on_sc=False),
    )
    def gather_kernel(t_hbm, table_hbm, out_hbm, idx_v, rows_v, sem):
        wid = lax.axis_index("s") * info.num_cores + lax.axis_index("c")
        base = wid * n_per_w
        pltpu.sync_copy(t_hbm.at[pl.ds(base, n_per_w)], idx_v)
        for i in range(n_per_w // 16):
            sl = pl.ds(i * 16, 16)
            idx_v[sl] = lax.rem(idx_v[sl], jnp.int32(period))
        pltpu.async_copy(table_hbm.at[idx_v], rows_v, sem).wait()
        pltpu.sync_copy(rows_v, out_hbm.at[pl.ds(base, n_per_w)])

    return gather_kernel(t, pe_weight)


def _broadcast_add(img2d, pe2d):
    b, m = img2d.shape
    bb, lb = 8, 32768

    def body(img_ref, pe_ref, out_ref):
        out_ref[...] = img_ref[...] + pe_ref[...]

    return pl.pallas_call(
        body,
        grid=(m // lb, b // bb),
        in_specs=[
            pl.BlockSpec((bb, lb), lambda j, i: (i, j)),
            pl.BlockSpec((1, lb), lambda j, i: (0, j)),
        ],
        out_specs=pl.BlockSpec((bb, lb), lambda j, i: (i, j)),
        out_shape=jax.ShapeDtypeStruct((b, m), jnp.float32),
    )(img2d, pe2d)


def kernel(image_embeds, t, pe_weight):
    b, n, c = image_embeds.shape
    pe = _gather_pe(t.astype(jnp.int32), pe_weight.astype(jnp.float32))
    img2d = image_embeds.reshape(b, n * c)
    pe2d = pe.reshape(1, n * c)
    out = _broadcast_add(img2d, pe2d)
    return out.reshape(b, n, c)


# speedup vs baseline: 1.0088x; 1.0088x over previous
import functools

import jax
import jax.numpy as jnp
from jax import lax
from jax.experimental import pallas as pl
from jax.experimental.pallas import tpu as pltpu
from jax.experimental.pallas import tpu_sc as plsc


def _gather_pe(t, pe_weight):
    (n,) = t.shape
    period, c = pe_weight.shape
    info = plsc.get_sparse_core_info()
    nw = info.num_cores * info.num_subcores
    n_per_w = n // nw
    mesh = plsc.VectorSubcoreMesh(core_axis_name="c", subcore_axis_name="s")

    @functools.partial(
        pl.kernel,
        mesh=mesh,
        out_type=jax.ShapeDtypeStruct((n, c), jnp.float32),
        scratch_types=[
            pltpu.VMEM((n_per_w,), jnp.int32),
            pltpu.VMEM((n_per_w, c), jnp.float32),
            pltpu.SemaphoreType.DMA,
        ],
        compiler_params=pltpu.CompilerParams(use_tc_tiling_on_sc=False),
    )
    def gather_kernel(t_hbm, table_hbm, out_hbm, idx_v, rows_v, sem):
        wid = lax.axis_index("s") * info.num_cores + lax.axis_index("c")
        base = wid * n_per_w
        pltpu.sync_copy(t_hbm.at[pl.ds(base, n_per_w)], idx_v)
        for i in range(n_per_w // 16):
            sl = pl.ds(i * 16, 16)
            idx_v[sl] = lax.rem(idx_v[sl], jnp.int32(period))
        pltpu.async_copy(table_hbm.at[idx_v], rows_v, sem).wait()
        pltpu.sync_copy(rows_v, out_hbm.at[pl.ds(base, n_per_w)])

    return gather_kernel(t, pe_weight)


def _broadcast_add(img, pe):
    b, n, c = img.shape
    bb, nb = 8, 512

    def body(img_ref, pe_ref, out_ref):
        out_ref[...] = img_ref[...] + pe_ref[...][None]

    return pl.pallas_call(
        body,
        grid=(n // nb, b // bb),
        in_specs=[
            pl.BlockSpec((bb, nb, c), lambda j, i: (i, j, 0)),
            pl.BlockSpec((nb, c), lambda j, i: (j, 0)),
        ],
        out_specs=pl.BlockSpec((bb, nb, c), lambda j, i: (i, j, 0)),
        out_shape=jax.ShapeDtypeStruct((b, n, c), jnp.float32),
    )(img, pe)


def kernel(image_embeds, t, pe_weight):
    pe = _gather_pe(t.astype(jnp.int32), pe_weight.astype(jnp.float32))
    return _broadcast_add(image_embeds, pe)


# speedup vs baseline: 1.0380x; 1.0290x over previous
import functools

import jax
import jax.numpy as jnp
from jax import lax
from jax.experimental import pallas as pl
from jax.experimental.pallas import tpu as pltpu
from jax.experimental.pallas import tpu_sc as plsc


def _gather_pe(t, pe_weight):
    (n,) = t.shape
    period, c = pe_weight.shape
    info = plsc.get_sparse_core_info()
    nw = info.num_cores * info.num_subcores
    n_per_w = n // nw
    mesh = plsc.VectorSubcoreMesh(core_axis_name="c", subcore_axis_name="s")

    @functools.partial(
        pl.kernel,
        mesh=mesh,
        out_type=jax.ShapeDtypeStruct((n, c), jnp.float32),
        scratch_types=[
            pltpu.VMEM((n_per_w,), jnp.int32),
            pltpu.VMEM((n_per_w, c), jnp.float32),
            pltpu.SemaphoreType.DMA,
        ],
        compiler_params=pltpu.CompilerParams(use_tc_tiling_on_sc=False),
    )
    def gather_kernel(t_hbm, table_hbm, out_hbm, idx_v, rows_v, sem):
        wid = lax.axis_index("s") * info.num_cores + lax.axis_index("c")
        base = wid * n_per_w
        pltpu.sync_copy(t_hbm.at[pl.ds(base, n_per_w)], idx_v)
        for i in range(n_per_w // 16):
            sl = pl.ds(i * 16, 16)
            idx_v[sl] = lax.rem(idx_v[sl], jnp.int32(period))
        pltpu.async_copy(table_hbm.at[idx_v], rows_v, sem).wait()
        pltpu.sync_copy(rows_v, out_hbm.at[pl.ds(base, n_per_w)])

    return gather_kernel(t, pe_weight)


def _broadcast_add(img, pe):
    b, n, c = img.shape
    bb, nb = 8, 2048

    def body(img_ref, pe_ref, out_ref):
        out_ref[...] = img_ref[...] + pe_ref[...][None]

    return pl.pallas_call(
        body,
        grid=(n // nb, b // bb),
        in_specs=[
            pl.BlockSpec((bb, nb, c), lambda j, i: (i, j, 0)),
            pl.BlockSpec((nb, c), lambda j, i: (j, 0)),
        ],
        out_specs=pl.BlockSpec((bb, nb, c), lambda j, i: (i, j, 0)),
        out_shape=jax.ShapeDtypeStruct((b, n, c), jnp.float32),
    )(img, pe)


def kernel(image_embeds, t, pe_weight):
    pe = _gather_pe(t.astype(jnp.int32), pe_weight.astype(jnp.float32))
    return _broadcast_add(image_embeds, pe)
